# baseline (speedup 1.0000x reference)
import functools

import jax
import jax.numpy as jnp
from jax import lax
from jax.experimental import pallas as pl
from jax.experimental.pallas import tpu as pltpu
from jax.experimental.pallas import tpu_sc as plsc

N = 10000
E = 160000
K = 20
NEMB = 16
NGRAPH = 100
NSTEM = 2000
NBLOCK = 106
NSTYPE = 21
OUTS = 105

NC = 2
NS = 16
NW = NC * NS
CL = 128
NCHUNK = 40
EPW = NCHUNK * CL
EPAD = NW * EPW

SLOTS = K * N
SPW = 12544
SPAD = NS * SPW
TRASH = SLOTS

_f32 = jnp.float32


def _lrelu(t):
    return jnp.where(t > 0, t, 0.01 * t)


_WIN = 16


def _edge_sc_body(p_hbm, gidx_hbm, sidx_hbm, zeros_hbm, s_out,
                  gidx_v, sidx_v, vals_v, zeros_v, p_shared, s_shared,
                  gsem, ssem):
    cid = lax.axis_index("c")
    sid = lax.axis_index("s")
    wid = cid * NS + sid
    pltpu.sync_copy(gidx_hbm.at[wid], gidx_v)
    pltpu.sync_copy(sidx_hbm.at[wid], sidx_v)
    poff = pl.multiple_of(sid * SPW, 128)
    pltpu.sync_copy(p_hbm.at[pl.ds(poff, SPW)], zeros_v)
    pltpu.sync_copy(zeros_v, p_shared.at[pl.ds(sid * SPW, SPW)])
    pltpu.sync_copy(zeros_hbm, zeros_v)
    pltpu.sync_copy(zeros_v, s_shared.at[pl.ds(sid * SPW, SPW)])
    plsc.subcore_barrier()

    def fire(j, carry):
        pltpu.async_copy(p_shared.at[gidx_v.at[j]], vals_v.at[j], gsem)
        return carry

    lax.fori_loop(0, _WIN, fire, 0)

    def chunk(j, carry):
        pltpu.make_async_copy(p_shared.at[gidx_v.at[j]], vals_v.at[j],
                              gsem).wait()
        pltpu.async_copy(vals_v.at[j], s_shared.at[sidx_v.at[j]], ssem,
                         add=True)

        @pl.when(j < NCHUNK - _WIN)
        def _():
            fire(j + _WIN, 0)

        return carry

    lax.fori_loop(0, NCHUNK, chunk, 0)

    def drain(j, carry):
        pltpu.make_async_copy(vals_v.at[j], s_shared.at[sidx_v.at[j]],
                              ssem).wait()
        return carry

    lax.fori_loop(0, NCHUNK, drain, 0)
    plsc.subcore_barrier()
    off = pl.multiple_of(cid * SPAD + sid * SPW, 128)
    pltpu.sync_copy(s_shared.at[pl.ds(sid * SPW, SPW)], zeros_v)
    pltpu.sync_copy(zeros_v, s_out.at[pl.ds(off, SPW)])


@functools.cache
def _build_edge_sc():
    mesh = plsc.VectorSubcoreMesh(core_axis_name="c", subcore_axis_name="s")
    return pl.kernel(
        _edge_sc_body,
        out_type=jax.ShapeDtypeStruct((NC * SPAD,), _f32),
        mesh=mesh,
        scratch_types=[
            pltpu.VMEM((NCHUNK, CL), jnp.int32),
            pltpu.VMEM((NCHUNK, CL), jnp.int32),
            pltpu.VMEM((NCHUNK, CL), _f32),
            pltpu.VMEM((SPW,), _f32),
            pltpu.VMEM_SHARED((SPAD,), _f32),
            pltpu.VMEM_SHARED((SPAD,), _f32),
            pltpu.SemaphoreType.DMA,
            pltpu.SemaphoreType.DMA,
        ],
    )


def _edge_sc(p_flat, gidx, sidx, zeros):
    p_pad = jnp.concatenate([p_flat, jnp.zeros((SPAD - SLOTS,), _f32)])
    return _build_edge_sc()(p_pad, gidx, sidx, zeros).reshape(NC, SPAD)


def _tc_embed_body(x_ref, batch_ref, cnt_ref, blockembT_ref, vecT_ref,
                   w1T_ref, b1_ref, w2T_ref, b2_ref, bondemb_ref,
                   outT_ref, pT_ref, invdeg_ref):
    x = x_ref[...]
    oh_x = (lax.broadcasted_iota(jnp.int32, (NBLOCK, N), 0) == x).astype(_f32)
    xeT = jnp.dot(blockembT_ref[...], oh_x, preferred_element_type=_f32)
    b = batch_ref[...]
    oh_b = (lax.broadcasted_iota(jnp.int32, (NGRAPH, N), 0) == b).astype(_f32)
    bvT = jnp.dot(vecT_ref[...], oh_b, preferred_element_type=_f32)
    cat = jnp.concatenate([xeT, bvT], axis=0)
    h1 = _lrelu(jnp.dot(w1T_ref[...], cat, preferred_element_type=_f32)
                + b1_ref[...])
    outT = jnp.dot(w2T_ref[...], h1, preferred_element_type=_f32) + b2_ref[...]
    outT_ref[...] = outT
    pT_ref[...] = jnp.dot(bondemb_ref[...], outT, preferred_element_type=_f32)
    deg = jnp.maximum(jnp.sum(cnt_ref[...], axis=0, keepdims=True), 1.0)
    invdeg_ref[...] = 1.0 / deg


_tc_embed = pl.pallas_call(
    _tc_embed_body,
    out_shape=[
        jax.ShapeDtypeStruct((NEMB, N), _f32),
        jax.ShapeDtypeStruct((K, N), _f32),
        jax.ShapeDtypeStruct((1, N), _f32),
    ],
)


def _tc_step_body(s_ref, outT_ref, invdeg_ref, bondembT_ref, bondemb_ref,
                  rootT_ref, cbias_ref, wihT_ref, bih_ref, whhT_ref, bhh_ref,
                  hT_ref, pT_ref):
    ST = s_ref[0] + s_ref[1]
    outT = outT_ref[...]
    hT = outT
    aggrT = jnp.dot(bondembT_ref[...], ST,
                    preferred_element_type=_f32) * invdeg_ref[...]
    m = _lrelu(aggrT
               + jnp.dot(rootT_ref[...], outT, preferred_element_type=_f32)
               + cbias_ref[...])
    gi = jnp.dot(wihT_ref[...], m, preferred_element_type=_f32) + bih_ref[...]
    gh = jnp.dot(whhT_ref[...], hT, preferred_element_type=_f32) + bhh_ref[...]
    r = jax.nn.sigmoid(gi[0:NEMB] + gh[0:NEMB])
    z = jax.nn.sigmoid(gi[NEMB:2 * NEMB] + gh[NEMB:2 * NEMB])
    n = jnp.tanh(gi[2 * NEMB:] + r * gh[2 * NEMB:])
    hT_new = (1.0 - z) * n + z * hT
    hT_ref[...] = hT_new
    pT_ref[...] = jnp.dot(bondemb_ref[...], hT_new, preferred_element_type=_f32)


_tc_step = pl.pallas_call(
    _tc_step_body,
    out_shape=[
        jax.ShapeDtypeStruct((NEMB, N), _f32),
        jax.ShapeDtypeStruct((K, N), _f32),
    ],
)


_GCH = 1000


def _tc_heads_body(outT_ref, sidx_ref, stype_ref, batch_ref, stemembT_ref,
                   sw1T_ref, sb1_ref, sw2T_ref, sb2_ref, sw3T_ref, sb3_ref,
                   gw1T_ref, gb1_ref, gw2T_ref, gb2_ref,
                   spT_ref, molT_ref):
    outT = outT_ref[...]
    sidx = sidx_ref[...]
    acc = jnp.zeros((NEMB, NSTEM), _f32)
    for c in range(N // _GCH):
        oh = (lax.broadcasted_iota(jnp.int32, (_GCH, NSTEM), 0) + c * _GCH
              == sidx).astype(_f32)
        acc = acc + jnp.dot(outT[:, c * _GCH:(c + 1) * _GCH], oh,
                            preferred_element_type=_f32)
    stype = stype_ref[...]
    oh_st = (lax.broadcasted_iota(jnp.int32, (NSTYPE, NSTEM), 0)
             == stype).astype(_f32)
    stT = jnp.dot(stemembT_ref[...], oh_st, preferred_element_type=_f32)
    cat = jnp.concatenate([acc, stT], axis=0)
    hs = _lrelu(jnp.dot(sw1T_ref[...], cat, preferred_element_type=_f32)
                + sb1_ref[...])
    hs = _lrelu(jnp.dot(sw2T_ref[...], hs, preferred_element_type=_f32)
                + sb2_ref[...])
    spT_ref[...] = (jnp.dot(sw3T_ref[...], hs, preferred_element_type=_f32)
                    + sb3_ref[...])

    bcol = batch_ref[...]
    oh_g = (lax.broadcasted_iota(jnp.int32, (N, NGRAPH), 1) == bcol).astype(_f32)
    gsumT = jnp.dot(outT, oh_g, preferred_element_type=_f32)
    gcnt = jnp.maximum(jnp.sum(oh_g, axis=0, keepdims=True), 1.0)
    gmeanT = gsumT / gcnt
    gh1 = _lrelu(jnp.dot(gw1T_ref[...], gmeanT, preferred_element_type=_f32)
                 + gb1_ref[...])
    molT_ref[...] = (jnp.dot(gw2T_ref[...], gh1, preferred_element_type=_f32)
                     + gb2_ref[...])


_tc_heads = pl.pallas_call(
    _tc_heads_body,
    out_shape=[
        jax.ShapeDtypeStruct((OUTS, NSTEM), _f32),
        jax.ShapeDtypeStruct((1, NGRAPH), _f32),
    ],
)


def kernel(x, edge_index, edge_attr_idx, stemtypes, batch, stems_batch, stems,
           slices_x, vec_data, blockemb, stememb, bondemb, conv_root,
           conv_bias, b2e_w1, b2e_b1, b2e_w2, b2e_b2, gru_wih, gru_bih,
           gru_whh, gru_bhh, s_w1, s_b1, s_w2, s_b2, s_w3, s_b3, g_w1, g_b1,
           g_w2, g_b2):
    src = edge_index[0].astype(jnp.int32)
    dst = edge_index[1].astype(jnp.int32)
    a0 = edge_attr_idx[:, 0].astype(jnp.int32)
    a1 = edge_attr_idx[:, 1].astype(jnp.int32)

    gidx = a0 * N + src
    sidx = a1 * N + dst
    pad = EPAD - E
    gidx = jnp.concatenate([gidx, jnp.zeros((pad,), jnp.int32)])
    sidx = jnp.concatenate([sidx, jnp.full((pad,), TRASH, jnp.int32)])
    gidx = gidx.reshape(NW, NCHUNK, CL)
    sidx = sidx.reshape(NW, NCHUNK, CL)
    zeros = jnp.zeros((SPW,), _f32)

    x2 = x.astype(jnp.int32).reshape(1, N)
    batch2 = batch.astype(jnp.int32).reshape(1, N)
    batch_col = batch.astype(jnp.int32).reshape(N, 1)

    cnt = _edge_sc(jnp.ones((SLOTS,), _f32), gidx, sidx, zeros)
    cnt40 = cnt[:, :SLOTS].reshape(NC * K, N)

    outT, pT, invdeg = _tc_embed(
        x2, batch2, cnt40, blockemb.T, vec_data.T,
        b2e_w1.T, b2e_b1.reshape(NEMB, 1), b2e_w2.T, b2e_b2.reshape(NEMB, 1),
        bondemb)

    step_args = (invdeg, bondemb.T, bondemb, conv_root.T,
                 conv_bias.reshape(NEMB, 1), gru_wih.T,
                 gru_bih.reshape(3 * NEMB, 1), gru_whh.T,
                 gru_bhh.reshape(3 * NEMB, 1))
    for _ in range(10):
        s_acc = _edge_sc(pT.reshape(SLOTS), gidx, sidx, zeros)
        s3 = s_acc[:, :SLOTS].reshape(NC, K, N)
        outT, pT = _tc_step(s3, outT, *step_args)

    stem_idx = (jnp.take(slices_x, stems_batch) + stems[:, 0]).astype(
        jnp.int32).reshape(1, NSTEM)
    stype2 = stemtypes.astype(jnp.int32).reshape(1, NSTEM)
    spT, molT = _tc_heads(
        outT, stem_idx, stype2, batch_col, stememb.T,
        s_w1.T, s_b1.reshape(NEMB, 1), s_w2.T, s_b2.reshape(NEMB, 1),
        s_w3.T, s_b3.reshape(OUTS, 1),
        g_w1.T, g_b1.reshape(NEMB, 1), g_w2.T, g_b2.reshape(1, 1))
    return spT.T, molT.T


# speedup vs baseline: 1.0059x; 1.0059x over previous
import functools

import jax
import jax.numpy as jnp
from jax import lax
from jax.experimental import pallas as pl
from jax.experimental.pallas import tpu as pltpu
from jax.experimental.pallas import tpu_sc as plsc

N = 10000
E = 160000
K = 20
NEMB = 16
NGRAPH = 100
NSTEM = 2000
NBLOCK = 106
NSTYPE = 21
OUTS = 105

NC = 2
NS = 16
NW = NC * NS
CL = 128
NCHUNK = 40
EPW = NCHUNK * CL
EPAD = NW * EPW

SLOTS = K * N
SPW = 12544
SPAD = NS * SPW
TRASH = SLOTS

_f32 = jnp.float32


def _lrelu(t):
    return jnp.where(t > 0, t, 0.01 * t)


_WIN = 8


def _edge_sc_body(p_hbm, gidx_hbm, sidx_hbm, zeros_hbm, s_out,
                  gidx_v, sidx_v, vals_v, zeros_v, p_shared, s_shared,
                  gsem, ssem):
    cid = lax.axis_index("c")
    sid = lax.axis_index("s")
    wid = cid * NS + sid
    pltpu.sync_copy(gidx_hbm.at[wid], gidx_v)
    pltpu.sync_copy(sidx_hbm.at[wid], sidx_v)
    poff = pl.multiple_of(sid * SPW, 128)
    pltpu.sync_copy(p_hbm.at[pl.ds(poff, SPW)], zeros_v)
    pltpu.sync_copy(zeros_v, p_shared.at[pl.ds(sid * SPW, SPW)])
    pltpu.sync_copy(zeros_hbm, zeros_v)
    pltpu.sync_copy(zeros_v, s_shared.at[pl.ds(sid * SPW, SPW)])
    plsc.subcore_barrier()

    def fire(j, carry):
        pltpu.async_copy(p_shared.at[gidx_v.at[j]], vals_v.at[j], gsem)
        return carry

    lax.fori_loop(0, _WIN, fire, 0)

    def chunk(j, carry):
        pltpu.make_async_copy(p_shared.at[gidx_v.at[j]], vals_v.at[j],
                              gsem).wait()
        pltpu.async_copy(vals_v.at[j], s_shared.at[sidx_v.at[j]], ssem,
                         add=True)

        @pl.when(j < NCHUNK - _WIN)
        def _():
            fire(j + _WIN, 0)

        return carry

    lax.fori_loop(0, NCHUNK, chunk, 0)

    def drain(j, carry):
        pltpu.make_async_copy(vals_v.at[j], s_shared.at[sidx_v.at[j]],
                              ssem).wait()
        return carry

    lax.fori_loop(0, NCHUNK, drain, 0)
    plsc.subcore_barrier()
    off = pl.multiple_of(cid * SPAD + sid * SPW, 128)
    pltpu.sync_copy(s_shared.at[pl.ds(sid * SPW, SPW)], zeros_v)
    pltpu.sync_copy(zeros_v, s_out.at[pl.ds(off, SPW)])


@functools.cache
def _build_edge_sc():
    mesh = plsc.VectorSubcoreMesh(core_axis_name="c", subcore_axis_name="s")
    return pl.kernel(
        _edge_sc_body,
        out_type=jax.ShapeDtypeStruct((NC * SPAD,), _f32),
        mesh=mesh,
        scratch_types=[
            pltpu.VMEM((NCHUNK, CL), jnp.int32),
            pltpu.VMEM((NCHUNK, CL), jnp.int32),
            pltpu.VMEM((NCHUNK, CL), _f32),
            pltpu.VMEM((SPW,), _f32),
            pltpu.VMEM_SHARED((SPAD,), _f32),
            pltpu.VMEM_SHARED((SPAD,), _f32),
            pltpu.SemaphoreType.DMA,
            pltpu.SemaphoreType.DMA,
        ],
    )


def _edge_sc(p_flat, gidx, sidx, zeros):
    p_pad = jnp.concatenate([p_flat, jnp.zeros((SPAD - SLOTS,), _f32)])
    return _build_edge_sc()(p_pad, gidx, sidx, zeros).reshape(NC, SPAD)


def _tc_embed_body(x_ref, batch_ref, cnt_ref, blockembT_ref, vecT_ref,
                   w1T_ref, b1_ref, w2T_ref, b2_ref, bondemb_ref,
                   outT_ref, pT_ref, invdeg_ref):
    x = x_ref[...]
    oh_x = (lax.broadcasted_iota(jnp.int32, (NBLOCK, N), 0) == x).astype(_f32)
    xeT = jnp.dot(blockembT_ref[...], oh_x, preferred_element_type=_f32)
    b = batch_ref[...]
    oh_b = (lax.broadcasted_iota(jnp.int32, (NGRAPH, N), 0) == b).astype(_f32)
    bvT = jnp.dot(vecT_ref[...], oh_b, preferred_element_type=_f32)
    cat = jnp.concatenate([xeT, bvT], axis=0)
    h1 = _lrelu(jnp.dot(w1T_ref[...], cat, preferred_element_type=_f32)
                + b1_ref[...])
    outT = jnp.dot(w2T_ref[...], h1, preferred_element_type=_f32) + b2_ref[...]
    outT_ref[...] = outT
    pT_ref[...] = jnp.dot(bondemb_ref[...], outT, preferred_element_type=_f32)
    deg = jnp.maximum(jnp.sum(cnt_ref[...], axis=0, keepdims=True), 1.0)
    invdeg_ref[...] = 1.0 / deg


_tc_embed = pl.pallas_call(
    _tc_embed_body,
    out_shape=[
        jax.ShapeDtypeStruct((NEMB, N), _f32),
        jax.ShapeDtypeStruct((K, N), _f32),
        jax.ShapeDtypeStruct((1, N), _f32),
    ],
)


def _tc_step_body(s_ref, outT_ref, invdeg_ref, bondembT_ref, bondemb_ref,
                  rootT_ref, cbias_ref, wihT_ref, bih_ref, whhT_ref, bhh_ref,
                  hT_ref, pT_ref):
    ST = s_ref[0] + s_ref[1]
    outT = outT_ref[...]
    hT = outT
    aggrT = jnp.dot(bondembT_ref[...], ST,
                    preferred_element_type=_f32) * invdeg_ref[...]
    m = _lrelu(aggrT
               + jnp.dot(rootT_ref[...], outT, preferred_element_type=_f32)
               + cbias_ref[...])
    gi = jnp.dot(wihT_ref[...], m, preferred_element_type=_f32) + bih_ref[...]
    gh = jnp.dot(whhT_ref[...], hT, preferred_element_type=_f32) + bhh_ref[...]
    r = jax.nn.sigmoid(gi[0:NEMB] + gh[0:NEMB])
    z = jax.nn.sigmoid(gi[NEMB:2 * NEMB] + gh[NEMB:2 * NEMB])
    n = jnp.tanh(gi[2 * NEMB:] + r * gh[2 * NEMB:])
    hT_new = (1.0 - z) * n + z * hT
    hT_ref[...] = hT_new
    pT_ref[...] = jnp.dot(bondemb_ref[...], hT_new, preferred_element_type=_f32)


_tc_step = pl.pallas_call(
    _tc_step_body,
    out_shape=[
        jax.ShapeDtypeStruct((NEMB, N), _f32),
        jax.ShapeDtypeStruct((K, N), _f32),
    ],
)


_GCH = 1000


def _tc_heads_body(outT_ref, sidx_ref, stype_ref, batch_ref, stemembT_ref,
                   sw1T_ref, sb1_ref, sw2T_ref, sb2_ref, sw3T_ref, sb3_ref,
                   gw1T_ref, gb1_ref, gw2T_ref, gb2_ref,
                   spT_ref, molT_ref):
    outT = outT_ref[...]
    sidx = sidx_ref[...]
    acc = jnp.zeros((NEMB, NSTEM), _f32)
    for c in range(N // _GCH):
        oh = (lax.broadcasted_iota(jnp.int32, (_GCH, NSTEM), 0) + c * _GCH
              == sidx).astype(_f32)
        acc = acc + jnp.dot(outT[:, c * _GCH:(c + 1) * _GCH], oh,
                            preferred_element_type=_f32)
    stype = stype_ref[...]
    oh_st = (lax.broadcasted_iota(jnp.int32, (NSTYPE, NSTEM), 0)
             == stype).astype(_f32)
    stT = jnp.dot(stemembT_ref[...], oh_st, preferred_element_type=_f32)
    cat = jnp.concatenate([acc, stT], axis=0)
    hs = _lrelu(jnp.dot(sw1T_ref[...], cat, preferred_element_type=_f32)
                + sb1_ref[...])
    hs = _lrelu(jnp.dot(sw2T_ref[...], hs, preferred_element_type=_f32)
                + sb2_ref[...])
    spT_ref[...] = (jnp.dot(sw3T_ref[...], hs, preferred_element_type=_f32)
                    + sb3_ref[...])

    bcol = batch_ref[...]
    oh_g = (lax.broadcasted_iota(jnp.int32, (N, NGRAPH), 1) == bcol).astype(_f32)
    gsumT = jnp.dot(outT, oh_g, preferred_element_type=_f32)
    gcnt = jnp.maximum(jnp.sum(oh_g, axis=0, keepdims=True), 1.0)
    gmeanT = gsumT / gcnt
    gh1 = _lrelu(jnp.dot(gw1T_ref[...], gmeanT, preferred_element_type=_f32)
                 + gb1_ref[...])
    molT_ref[...] = (jnp.dot(gw2T_ref[...], gh1, preferred_element_type=_f32)
                     + gb2_ref[...])


_tc_heads = pl.pallas_call(
    _tc_heads_body,
    out_shape=[
        jax.ShapeDtypeStruct((OUTS, NSTEM), _f32),
        jax.ShapeDtypeStruct((1, NGRAPH), _f32),
    ],
)


def kernel(x, edge_index, edge_attr_idx, stemtypes, batch, stems_batch, stems,
           slices_x, vec_data, blockemb, stememb, bondemb, conv_root,
           conv_bias, b2e_w1, b2e_b1, b2e_w2, b2e_b2, gru_wih, gru_bih,
           gru_whh, gru_bhh, s_w1, s_b1, s_w2, s_b2, s_w3, s_b3, g_w1, g_b1,
           g_w2, g_b2):
    src = edge_index[0].astype(jnp.int32)
    dst = edge_index[1].astype(jnp.int32)
    a0 = edge_attr_idx[:, 0].astype(jnp.int32)
    a1 = edge_attr_idx[:, 1].astype(jnp.int32)

    gidx = a0 * N + src
    sidx = a1 * N + dst
    pad = EPAD - E
    gidx = jnp.concatenate([gidx, jnp.zeros((pad,), jnp.int32)])
    sidx = jnp.concatenate([sidx, jnp.full((pad,), TRASH, jnp.int32)])
    gidx = gidx.reshape(NW, NCHUNK, CL)
    sidx = sidx.reshape(NW, NCHUNK, CL)
    zeros = jnp.zeros((SPW,), _f32)

    x2 = x.astype(jnp.int32).reshape(1, N)
    batch2 = batch.astype(jnp.int32).reshape(1, N)
    batch_col = batch.astype(jnp.int32).reshape(N, 1)

    cnt = _edge_sc(jnp.ones((SLOTS,), _f32), gidx, sidx, zeros)
    cnt40 = cnt[:, :SLOTS].reshape(NC * K, N)

    outT, pT, invdeg = _tc_embed(
        x2, batch2, cnt40, blockemb.T, vec_data.T,
        b2e_w1.T, b2e_b1.reshape(NEMB, 1), b2e_w2.T, b2e_b2.reshape(NEMB, 1),
        bondemb)

    step_args = (invdeg, bondemb.T, bondemb, conv_root.T,
                 conv_bias.reshape(NEMB, 1), gru_wih.T,
                 gru_bih.reshape(3 * NEMB, 1), gru_whh.T,
                 gru_bhh.reshape(3 * NEMB, 1))
    for _ in range(10):
        s_acc = _edge_sc(pT.reshape(SLOTS), gidx, sidx, zeros)
        s3 = s_acc[:, :SLOTS].reshape(NC, K, N)
        outT, pT = _tc_step(s3, outT, *step_args)

    stem_idx = (jnp.take(slices_x, stems_batch) + stems[:, 0]).astype(
        jnp.int32).reshape(1, NSTEM)
    stype2 = stemtypes.astype(jnp.int32).reshape(1, NSTEM)
    spT, molT = _tc_heads(
        outT, stem_idx, stype2, batch_col, stememb.T,
        s_w1.T, s_b1.reshape(NEMB, 1), s_w2.T, s_b2.reshape(NEMB, 1),
        s_w3.T, s_b3.reshape(OUTS, 1),
        g_w1.T, g_b1.reshape(NEMB, 1), g_w2.T, g_b2.reshape(1, 1))
    return spT.T, molT.T
